# baseline (speedup 1.0000x reference)
import functools
import math

import jax
import jax.numpy as jnp
from jax import lax
from jax.experimental import pallas as pl
from jax.experimental.pallas import tpu as pltpu
from jax.experimental.pallas import tpu_sc as plsc

D_MODEL = 64
VOCAB = 1000000
SCALE = math.sqrt(D_MODEL)
LANES = 16

NUM_CORES = 2
NUM_SUBCORES = 16
NUM_WORKERS = NUM_CORES * NUM_SUBCORES

PACK_LINES = VOCAB // 2

PACK_CHUNK = 384
PACK_FULL_CHUNKS = VOCAB // PACK_CHUNK
PACK_TAIL_START = PACK_FULL_CHUNKS * PACK_CHUNK
PACK_TAIL = VOCAB - PACK_TAIL_START
PACK_ITERS = -(-PACK_FULL_CHUNKS // NUM_WORKERS)

SEQ = 50
BATCH = 16384
CB = 256
N_BLOCKS = BATCH // CB
BLOCKS_PER_W = N_BLOCKS // NUM_WORKERS

_MESH = dict(core_axis_name="c", subcore_axis_name="s",
             num_cores=NUM_CORES, num_subcores=NUM_SUBCORES)


def _wid():
    return lax.axis_index("s") * NUM_CORES + lax.axis_index("c")


def _pack_kernel():
    mesh = plsc.VectorSubcoreMesh(**_MESH)

    @functools.partial(
        pl.kernel,
        out_type=jax.ShapeDtypeStruct((PACK_LINES, 128), jnp.float32),
        mesh=mesh,
        scratch_types=[
            pltpu.VMEM((D_MODEL, PACK_CHUNK), jnp.float32),
            pltpu.VMEM((PACK_CHUNK // 2, 128), jnp.float32),
            pltpu.VMEM((D_MODEL, PACK_TAIL), jnp.float32),
        ],
        compiler_params=pltpu.CompilerParams(use_tc_tiling_on_sc=True,
                                             needs_layout_passes=False),
    )
    def body(lutT_hbm, packed_hbm, src_v, stg_v, tail_v):
        w = _wid()
        iota = lax.iota(jnp.int32, LANES)

        def rearrange(src_ref, stg_ref, r_len):
            def c_loop(c, _):
                def j_loop(j, _):
                    r16 = j * LANES
                    vals = src_ref[c, pl.ds(r16, LANES)]
                    rv = iota + r16
                    lv = lax.shift_right_logical(rv, 1)
                    cv = lax.bitwise_and(rv, 1) * D_MODEL + c
                    plsc.store_scatter(stg_ref, [lv, cv], vals)
                    return 0
                lax.fori_loop(0, r_len // LANES, j_loop, 0, unroll=4)
                return 0
            lax.fori_loop(0, D_MODEL, c_loop, 0)

        def k_loop(k, _):
            g = w + k * NUM_WORKERS

            @pl.when(g < PACK_FULL_CHUNKS)
            def _():
                r0 = pl.multiple_of(g * PACK_CHUNK, PACK_CHUNK)
                l0 = pl.multiple_of(g * (PACK_CHUNK // 2), PACK_CHUNK // 2)
                pltpu.sync_copy(lutT_hbm.at[:, pl.ds(r0, PACK_CHUNK)], src_v)
                rearrange(src_v, stg_v, PACK_CHUNK)
                pltpu.sync_copy(stg_v,
                                packed_hbm.at[pl.ds(l0, PACK_CHUNK // 2)])
            return 0

        lax.fori_loop(0, PACK_ITERS, k_loop, 0)

        @pl.when(w == NUM_WORKERS - 1)
        def _():
            pltpu.sync_copy(lutT_hbm.at[:, pl.ds(PACK_TAIL_START, PACK_TAIL)],
                            tail_v)
            rearrange(tail_v, stg_v, PACK_TAIL)
            pltpu.sync_copy(stg_v.at[pl.ds(0, PACK_TAIL // 2)],
                            packed_hbm.at[pl.ds(PACK_TAIL_START // 2,
                                                PACK_TAIL // 2)])

    return body


def _gather_kernel():
    mesh = plsc.VectorSubcoreMesh(**_MESH)

    @functools.partial(
        pl.kernel,
        out_type=jax.ShapeDtypeStruct((SEQ, D_MODEL, BATCH), jnp.float32),
        mesh=mesh,
        scratch_types=[
            pltpu.VMEM((CB * SEQ,), jnp.int32),
            pltpu.VMEM((2, 128), jnp.int32),
            pltpu.VMEM((CB,), jnp.int32),
            pltpu.VMEM((CB, 128), jnp.float32),
            pltpu.VMEM((D_MODEL, CB), jnp.float32),
            pltpu.SemaphoreType.DMA,
        ],
        compiler_params=pltpu.CompilerParams(use_tc_tiling_on_sc=True,
                                             needs_layout_passes=False),
    )
    def body(xf_hbm, packed_hbm, out_hbm, xv, idx_v, h_v, rows_v, stg_v, sem):
        w = _wid()
        iota = lax.iota(jnp.int32, LANES)

        for bi in range(BLOCKS_PER_W):
            blk = w * BLOCKS_PER_W + bi
            b0 = pl.multiple_of(blk * CB, CB)
            pltpu.sync_copy(xf_hbm.at[pl.ds(b0 * SEQ, CB * SEQ)], xv)

            def s_loop(s, _):
                for bg in range(CB // LANES):
                    offs = iota * SEQ + (bg * LANES * SEQ + s)
                    v = plsc.load_gather(xv, [offs])
                    idx_v[bg // 8, pl.ds((bg % 8) * LANES, LANES)] = (
                        lax.shift_right_logical(v, 1))
                    h_v[pl.ds(bg * LANES, LANES)] = (
                        lax.bitwise_and(v, 1) * D_MODEL)

                cp0 = pltpu.async_copy(
                    packed_hbm.at[idx_v.at[0]], rows_v.at[pl.ds(0, 128)], sem)
                cp1 = pltpu.async_copy(
                    packed_hbm.at[idx_v.at[1]], rows_v.at[pl.ds(128, 128)], sem)
                cp0.wait()
                cp1.wait()

                for bg in range(CB // LANES):
                    bidx = iota + bg * LANES
                    hv = h_v[pl.ds(bg * LANES, LANES)]

                    def d_loop(d, _):
                        vals = plsc.load_gather(rows_v, [bidx, hv + d])
                        stg_v[d, pl.ds(bg * LANES, LANES)] = vals * SCALE
                        return 0
                    lax.fori_loop(0, D_MODEL, d_loop, 0, unroll=8)

                pltpu.sync_copy(stg_v, out_hbm.at[s, :, pl.ds(b0, CB)])
                return 0

            lax.fori_loop(0, SEQ, s_loop, 0)

    return body


def kernel(x, lut):
    xf = x.reshape(-1).astype(jnp.int32)
    packed = _pack_kernel()(lut.T)
    outT = _gather_kernel()(xf, packed)
    return jnp.transpose(outT, (2, 0, 1))


# speedup vs baseline: 1.8999x; 1.8999x over previous
import functools
import math

import jax
import jax.numpy as jnp
from jax import lax
from jax.experimental import pallas as pl
from jax.experimental.pallas import tpu as pltpu
from jax.experimental.pallas import tpu_sc as plsc

D_MODEL = 64
VOCAB = 1000000
SCALE = math.sqrt(D_MODEL)
LANES = 16

NUM_CORES = 2
NUM_SUBCORES = 16
NUM_WORKERS = NUM_CORES * NUM_SUBCORES

PACK_LINES = VOCAB // 2

PACK_CHUNK = 384
PACK_J = PACK_CHUNK // LANES
PACK_FULL_CHUNKS = VOCAB // PACK_CHUNK
PACK_TAIL_START = PACK_FULL_CHUNKS * PACK_CHUNK
PACK_TAIL = VOCAB - PACK_TAIL_START
PACK_ITERS = -(-PACK_FULL_CHUNKS // NUM_WORKERS)

SEQ = 50
BATCH = 16384
CB = 256
N_BLOCKS = BATCH // CB
BLOCKS_PER_W = N_BLOCKS // NUM_WORKERS

_MESH = dict(core_axis_name="c", subcore_axis_name="s",
             num_cores=NUM_CORES, num_subcores=NUM_SUBCORES)


def _wid():
    return lax.axis_index("s") * NUM_CORES + lax.axis_index("c")


def _pack_kernel():
    mesh = plsc.VectorSubcoreMesh(**_MESH)

    @functools.partial(
        pl.kernel,
        out_type=jax.ShapeDtypeStruct((PACK_LINES, 128), jnp.float32),
        mesh=mesh,
        scratch_types=[
            pltpu.VMEM((D_MODEL, PACK_CHUNK), jnp.float32),
            pltpu.VMEM((D_MODEL, PACK_CHUNK), jnp.float32),
            pltpu.VMEM((PACK_CHUNK // 2, 128), jnp.float32),
            pltpu.VMEM((PACK_CHUNK // 2, 128), jnp.float32),
            pltpu.VMEM((D_MODEL, PACK_TAIL), jnp.float32),
            pltpu.SemaphoreType.DMA,
            pltpu.SemaphoreType.DMA,
            pltpu.SemaphoreType.DMA,
            pltpu.SemaphoreType.DMA,
        ],
        compiler_params=pltpu.CompilerParams(use_tc_tiling_on_sc=True,
                                             needs_layout_passes=False),
    )
    def body(lutT_hbm, packed_hbm, src0, src1, stg0, stg1, tail_v,
             ld0, ld1, st0, st1):
        w = _wid()
        iota = lax.iota(jnp.int32, LANES)
        lv0 = lax.shift_right_logical(iota, 1)
        cvb = lax.bitwise_and(iota, 1) * D_MODEL

        def r_start(k, src, sem):
            g = lax.min(w + k * NUM_WORKERS, PACK_FULL_CHUNKS - 1)
            r0 = pl.multiple_of(g * PACK_CHUNK, PACK_CHUNK)
            pltpu.async_copy(lutT_hbm.at[:, pl.ds(r0, PACK_CHUNK)], src, sem)

        def r_wait(src, sem):
            pltpu.make_async_copy(lutT_hbm.at[:, pl.ds(0, PACK_CHUNK)],
                                  src, sem).wait()

        def w_start(k, stg, sem):
            g = lax.min(w + k * NUM_WORKERS, PACK_FULL_CHUNKS - 1)
            l0 = pl.multiple_of(g * (PACK_CHUNK // 2), PACK_CHUNK // 2)
            pltpu.async_copy(stg, packed_hbm.at[pl.ds(l0, PACK_CHUNK // 2)],
                             sem)

        def w_wait(stg, sem):
            pltpu.make_async_copy(stg,
                                  packed_hbm.at[pl.ds(0, PACK_CHUNK // 2)],
                                  sem).wait()

        def rearrange(src_ref, stg_ref, n_j):
            def c_body(c):
                cv = cvb + c
                for j in range(n_j):
                    vals = src_ref[c, pl.ds(j * LANES, LANES)]
                    plsc.store_scatter(stg_ref, [lv0 + 8 * j, cv], vals)
            plsc.parallel_loop(0, D_MODEL, 1, unroll=2)(c_body)

        r_start(0, src0, ld0)
        r_start(1, src1, ld1)

        def t_loop(t, _):
            k0 = 2 * t
            r_wait(src0, ld0)

            @pl.when(t > 0)
            def _():
                w_wait(stg0, st0)
            rearrange(src0, stg0, PACK_J)
            w_start(k0, stg0, st0)

            @pl.when(k0 + 2 < PACK_ITERS)
            def _():
                r_start(k0 + 2, src0, ld0)

            r_wait(src1, ld1)

            @pl.when(t > 0)
            def _():
                w_wait(stg1, st1)
            rearrange(src1, stg1, PACK_J)
            w_start(k0 + 1, stg1, st1)

            @pl.when(k0 + 3 < PACK_ITERS)
            def _():
                r_start(k0 + 3, src1, ld1)
            return 0

        lax.fori_loop(0, PACK_ITERS // 2, t_loop, 0)
        w_wait(stg0, st0)
        w_wait(stg1, st1)

        @pl.when(w == NUM_WORKERS - 1)
        def _():
            pltpu.sync_copy(lutT_hbm.at[:, pl.ds(PACK_TAIL_START, PACK_TAIL)],
                            tail_v)
            rearrange(tail_v, stg0, PACK_TAIL // LANES)
            pltpu.sync_copy(stg0.at[pl.ds(0, PACK_TAIL // 2)],
                            packed_hbm.at[pl.ds(PACK_TAIL_START // 2,
                                                PACK_TAIL // 2)])

    return body


def _gather_kernel():
    mesh = plsc.VectorSubcoreMesh(**_MESH)

    @functools.partial(
        pl.kernel,
        out_type=jax.ShapeDtypeStruct((SEQ, D_MODEL, BATCH), jnp.float32),
        mesh=mesh,
        scratch_types=[
            pltpu.VMEM((CB * SEQ,), jnp.int32),
            pltpu.VMEM((2, 128), jnp.int32),
            pltpu.VMEM((2, 128), jnp.int32),
            pltpu.VMEM((CB,), jnp.int32),
            pltpu.VMEM((CB,), jnp.int32),
            pltpu.VMEM((CB, 128), jnp.float32),
            pltpu.VMEM((CB, 128), jnp.float32),
            pltpu.VMEM((D_MODEL, CB), jnp.float32),
            pltpu.VMEM((D_MODEL, CB), jnp.float32),
            pltpu.SemaphoreType.DMA,
            pltpu.SemaphoreType.DMA,
            pltpu.SemaphoreType.DMA,
            pltpu.SemaphoreType.DMA,
        ],
        compiler_params=pltpu.CompilerParams(use_tc_tiling_on_sc=True,
                                             needs_layout_passes=False),
    )
    def body(xf_hbm, packed_hbm, out_hbm, xv, idx0, idx1, h0, h1,
             rows0, rows1, stg0, stg1, g0, g1, w0, w1):
        w = _wid()
        iota = lax.iota(jnp.int32, LANES)
        iota_seq = iota * SEQ

        def extract_and_fire(s, b0, idx_v, h_v, rows_v, sem):
            for bg in range(CB // LANES):
                offs = iota_seq + (bg * LANES * SEQ + s)
                v = plsc.load_gather(xv, [offs])
                idx_v[bg // 8, pl.ds((bg % 8) * LANES, LANES)] = (
                    lax.shift_right_logical(v, 1))
                h_v[pl.ds(bg * LANES, LANES)] = (
                    lax.bitwise_and(v, 1) * D_MODEL)
            pltpu.async_copy(packed_hbm.at[idx_v.at[0]],
                             rows_v.at[pl.ds(0, 128)], sem)
            pltpu.async_copy(packed_hbm.at[idx_v.at[1]],
                             rows_v.at[pl.ds(128, 128)], sem)

        def g_wait(idx_v, rows_v, sem):
            pltpu.make_async_copy(packed_hbm.at[idx_v.at[0]],
                                  rows_v.at[pl.ds(0, 128)], sem).wait()
            pltpu.make_async_copy(packed_hbm.at[idx_v.at[1]],
                                  rows_v.at[pl.ds(128, 128)], sem).wait()

        def rearrange(rows_v, h_v, stg_v):
            for bg in range(CB // LANES):
                bidx = iota + bg * LANES
                hv = h_v[pl.ds(bg * LANES, LANES)]

                def d_body(d):
                    vals = plsc.load_gather(rows_v, [bidx, hv + d])
                    stg_v[d, pl.ds(bg * LANES, LANES)] = vals * SCALE
                plsc.parallel_loop(0, D_MODEL, 1, unroll=4)(d_body)

        def w_start(s, b0, stg_v, sem):
            pltpu.async_copy(stg_v, out_hbm.at[s, :, pl.ds(b0, CB)], sem)

        def w_wait(b0, stg_v, sem):
            pltpu.make_async_copy(stg_v, out_hbm.at[0, :, pl.ds(b0, CB)],
                                  sem).wait()

        for bi in range(BLOCKS_PER_W):
            blk = w * BLOCKS_PER_W + bi
            b0 = pl.multiple_of(blk * CB, CB)
            pltpu.sync_copy(xf_hbm.at[pl.ds(b0 * SEQ, CB * SEQ)], xv)

            extract_and_fire(0, b0, idx0, h0, rows0, g0)
            extract_and_fire(1, b0, idx1, h1, rows1, g1)

            def s_loop(t, _):
                s0 = 2 * t
                g_wait(idx0, rows0, g0)

                @pl.when(t > 0)
                def _():
                    w_wait(b0, stg0, w0)
                rearrange(rows0, h0, stg0)
                w_start(s0, b0, stg0, w0)

                @pl.when(s0 + 2 < SEQ)
                def _():
                    extract_and_fire(s0 + 2, b0, idx0, h0, rows0, g0)

                g_wait(idx1, rows1, g1)

                @pl.when(t > 0)
                def _():
                    w_wait(b0, stg1, w1)
                rearrange(rows1, h1, stg1)
                w_start(s0 + 1, b0, stg1, w1)

                @pl.when(s0 + 3 < SEQ)
                def _():
                    extract_and_fire(s0 + 3, b0, idx1, h1, rows1, g1)
                return 0

            lax.fori_loop(0, SEQ // 2, s_loop, 0)
            w_wait(b0, stg0, w0)
            w_wait(b0, stg1, w1)

    return body


def kernel(x, lut):
    xf = x.reshape(-1).astype(jnp.int32)
    packed = _pack_kernel()(lut.T)
    outT = _gather_kernel()(xf, packed)
    return jnp.transpose(outT, (2, 0, 1))
